# baseline (speedup 1.0000x reference)
import functools

import jax
import jax.numpy as jnp
from jax import lax
from jax.experimental import pallas as pl
from jax.experimental.pallas import tpu as pltpu
from jax.experimental.pallas import tpu_sc as plsc

N = 10000
E = 320000
D = 128
NP = 10240
NPT = NP // 16
ROWS = 2560
EP = ROWS * 128
RC = ROWS // 16
RA = ROWS // 32

_MESH = plsc.VectorSubcoreMesh(
    core_axis_name="c", subcore_axis_name="s", num_cores=2, num_subcores=16
)


@functools.partial(
    pl.kernel,
    out_type=jax.ShapeDtypeStruct((2 * NP, D), jnp.float32),
    mesh=_MESH,
    scratch_types=[
        pltpu.VMEM_SHARED((NP, D), jnp.float32),
        pltpu.VMEM((128, D), jnp.float32),
        pltpu.VMEM((RA, 128), jnp.int32),
    ],
)
def _deg_kernel(dst_hbm, degp_hbm, hist, buf, dst_v):
    cid = lax.axis_index("c")
    sid = lax.axis_index("s")
    wid = sid * 2 + cid

    def zfill(r, _):
        for k in range(8):
            buf[r, pl.ds(k * 16, 16)] = jnp.zeros((16,), jnp.float32)
        return _

    lax.fori_loop(0, 128, zfill, None)
    for j in range(NPT // 128):
        pltpu.sync_copy(buf, hist.at[pl.ds(sid * NPT + j * 128, 128)])

    def ofill(r, _):
        for k in range(8):
            buf[r, pl.ds(k * 16, 16)] = jnp.ones((16,), jnp.float32)
        return _

    lax.fori_loop(0, 128, ofill, None)
    pltpu.sync_copy(dst_hbm.at[pl.ds(wid * RA, RA)], dst_v)
    plsc.subcore_barrier()

    def body(j, _):
        pltpu.sync_copy(buf, hist.at[dst_v.at[j]], add=True)
        return _

    lax.fori_loop(0, RA, body, None)
    plsc.subcore_barrier()
    obase = pl.multiple_of(cid * NP + sid * NPT, NPT)
    pltpu.sync_copy(hist.at[pl.ds(sid * NPT, NPT)],
                    degp_hbm.at[pl.ds(obase, NPT)])


def _prep_body(degp_ref, x0_ref, x1_ref, xs_ref):
    deg = degp_ref[0:N, :] + degp_ref[NP : NP + N, :]
    nl = lax.rsqrt(jnp.maximum(deg, 1.0))
    xs_ref[0:N, :] = x0_ref[...] * nl
    xs_ref[N : 2 * N, :] = x1_ref[...] * nl


def _prep_call(degp, x0, x1):
    return pl.pallas_call(
        _prep_body,
        out_shape=jax.ShapeDtypeStruct((2 * N, D), jnp.float32),
    )(degp, x0, x1)


@functools.partial(
    pl.kernel,
    out_type=jax.ShapeDtypeStruct((2 * NP, D), jnp.float32),
    mesh=_MESH,
    scratch_types=[
        pltpu.VMEM_SHARED((NP, D), jnp.float32),
        pltpu.VMEM((2, 16, 128), jnp.int32),
        pltpu.VMEM((2, 16, 128), jnp.int32),
        pltpu.VMEM((128, D), jnp.float32),
        pltpu.VMEM((128, D), jnp.float32),
    ]
    + [pltpu.SemaphoreType.DMA] * 16,
)
def _agg_kernel(xs_hbm, src_hbm, dst_hbm, agg_hbm,
                agg, src_v, dst_v, rows0, rows1, *sems):
    cid = lax.axis_index("c")
    sid = lax.axis_index("s")
    NCH = RC // 16
    GS = sems[:8]
    TS = sems[8:]

    def zfill(r, _):
        for k in range(8):
            rows0[r, pl.ds(k * 16, 16)] = jnp.zeros((16,), jnp.float32)
        return _

    lax.fori_loop(0, 128, zfill, None)
    for j in range(NPT // 128):
        pltpu.sync_copy(rows0, agg.at[pl.ds(sid * NPT + j * 128, 128)])
    plsc.subcore_barrier()

    tbase = sid * RC

    def stage(slot, c):
        b = pl.multiple_of(tbase + c * 16, 16)
        pltpu.sync_copy(src_hbm.at[pl.ds(cid * ROWS + b, 16)], src_v.at[slot])
        pltpu.sync_copy(dst_hbm.at[pl.ds(b, 16)], dst_v.at[slot])

    def slot_buf(s):
        return (rows0 if s < 4 else rows1).at[pl.ds((s % 4) * 32, 32)]

    def issue(islot, lr, qf, s):
        pltpu.async_copy(xs_hbm.at[src_v.at[islot, lr, pl.ds(qf * 32, 32)]],
                         slot_buf(s), GS[s])

    def wait_g(s):
        pltpu.make_async_copy(xs_hbm.at[pl.ds(0, 32)], slot_buf(s),
                              GS[s]).wait()

    def scat(islot, lr, qf, s):
        pltpu.async_copy(slot_buf(s),
                         agg.at[dst_v.at[islot, lr, pl.ds(qf * 32, 32)]],
                         TS[s], add=True)

    def wait_t(s):
        pltpu.make_async_copy(slot_buf(s), agg.at[pl.ds(0, 32)],
                              TS[s]).wait()

    stage(0, 0)
    for t in range(5):
        issue(0, t // 4, t % 4, t)

    for c in range(NCH):
        p, q = c % 2, (c + 1) % 2

        for k in range(8):
            t, s = k + 5, (k + 5) % 8
            if not (c == 0 and k < 3):
                wait_t(s)
            issue(p, t // 4, t % 4, s)
            wait_g(k)
            scat(p, k // 4, k % 4, k)

        if c < NCH - 1:
            stage(q, c + 1)

        def block(i, _, p=p):
            for k in range(8):
                s = (k + 5) % 8
                wait_t(s)
                issue(p, 2 * i + (k + 5) // 4, (k + 5) % 4, s)
                wait_g(k)
                scat(p, 2 * i + k // 4, k % 4, k)
            return _

        lax.fori_loop(1, 7, block, None)
        for k in range(8):
            t = 56 + k + 5
            s = t % 8
            if t < 64:
                wait_t(s)
                issue(p, 15, t % 4, s)
            elif c < NCH - 1:
                wait_t(s)
                issue(q, (t - 64) // 4, (t - 64) % 4, s)
            wait_g(k)
            scat(p, 14 + k // 4, k % 4, k)

    for s in range(8):
        wait_t(s)
    plsc.subcore_barrier()
    obase = pl.multiple_of(cid * NP + sid * NPT, NPT)
    pltpu.sync_copy(agg.at[pl.ds(sid * NPT, NPT)],
                    agg_hbm.at[pl.ds(obase, NPT)])


def _final_body(degp_ref, agg_ref, wl_ref, bl_ref, wc_ref, bc_ref, out_ref):
    deg = degp_ref[0:N, :] + degp_ref[NP : NP + N, :]
    nl = lax.rsqrt(jnp.maximum(deg, 1.0))
    a0 = agg_ref[0:N, :] * nl
    a1 = agg_ref[NP : NP + N, :] * nl
    r0 = jnp.dot(a0, wl_ref[...], preferred_element_type=jnp.float32)
    r0 = jnp.maximum(r0 + bl_ref[...][None, :], 0.0)
    r1 = jnp.dot(a1, wc_ref[...], preferred_element_type=jnp.float32)
    r1 = jnp.maximum(r1 + bc_ref[...][None, :], 0.0)
    out_ref[...] = r0 + r1


def _final_call(degp, agg, W_l, b_l, W_c, b_c):
    return pl.pallas_call(
        _final_body,
        out_shape=jax.ShapeDtypeStruct((N, D), jnp.float32),
    )(degp, agg, W_l, b_l, W_c, b_c)


def kernel(t, x, edge_index, W_l, b_l, W_c, b_c):
    del t
    x0 = x[:, :, 0]
    x1 = x[:, :, 1]
    ei = edge_index.astype(jnp.int32)
    pad = EP - E
    ar = jnp.arange(pad, dtype=jnp.int32)
    srcp = jnp.concatenate([ei[0], ar % N]).reshape(ROWS, 128)
    dstp = jnp.concatenate([ei[1], N + (ar % (NP - N))]).reshape(ROWS, 128)
    src_all = jnp.concatenate([srcp, srcp + N])

    degp = _deg_kernel(dstp)
    xs = _prep_call(degp, x0, x1)
    agg = _agg_kernel(xs, src_all, dstp)
    return _final_call(degp, agg, W_l, b_l, W_c, b_c)


# speedup vs baseline: 1.0099x; 1.0099x over previous
import functools

import jax
import jax.numpy as jnp
from jax import lax
from jax.experimental import pallas as pl
from jax.experimental.pallas import tpu as pltpu
from jax.experimental.pallas import tpu_sc as plsc

N = 10000
E = 320000
D = 128
NP = 10240
NPT = NP // 16
ROWS = 2560
EP = ROWS * 128
RC = ROWS // 16
RA = ROWS // 32

_MESH = plsc.VectorSubcoreMesh(
    core_axis_name="c", subcore_axis_name="s", num_cores=2, num_subcores=16
)


@functools.partial(
    pl.kernel,
    out_type=jax.ShapeDtypeStruct((2 * NP, D), jnp.float32),
    mesh=_MESH,
    scratch_types=[
        pltpu.VMEM_SHARED((NP, D), jnp.float32),
        pltpu.VMEM((128, D), jnp.float32),
        pltpu.VMEM((RA, 128), jnp.int32),
        pltpu.SemaphoreType.DMA,
    ],
)
def _deg_kernel(dst_hbm, degp_hbm, hist, buf, dst_v, sem):
    cid = lax.axis_index("c")
    sid = lax.axis_index("s")
    wid = sid * 2 + cid

    def zfill(r, _):
        for k in range(8):
            buf[r, pl.ds(k * 16, 16)] = jnp.zeros((16,), jnp.float32)
        return _

    lax.fori_loop(0, 128, zfill, None)
    for j in range(NPT // 128):
        pltpu.sync_copy(buf, hist.at[pl.ds(sid * NPT + j * 128, 128)])

    def ofill(r, _):
        for k in range(8):
            buf[r, pl.ds(k * 16, 16)] = jnp.ones((16,), jnp.float32)
        return _

    lax.fori_loop(0, 128, ofill, None)
    pltpu.sync_copy(dst_hbm.at[pl.ds(wid * RA, RA)], dst_v)
    plsc.subcore_barrier()

    def wait_one():
        pltpu.make_async_copy(buf, hist.at[pl.ds(0, 128)], sem).wait()

    for j in range(4):
        pltpu.async_copy(buf, hist.at[dst_v.at[j]], sem, add=True)

    def body(j, _):
        wait_one()
        pltpu.async_copy(buf, hist.at[dst_v.at[j]], sem, add=True)
        return _

    lax.fori_loop(4, RA, body, None)
    for _ in range(4):
        wait_one()
    plsc.subcore_barrier()
    obase = pl.multiple_of(cid * NP + sid * NPT, NPT)
    pltpu.sync_copy(hist.at[pl.ds(sid * NPT, NPT)],
                    degp_hbm.at[pl.ds(obase, NPT)])


def _prep_body(degp_ref, x0_ref, x1_ref, xs_ref):
    deg = degp_ref[0:N, :] + degp_ref[NP : NP + N, :]
    nl = lax.rsqrt(jnp.maximum(deg, 1.0))
    xs_ref[0:N, :] = x0_ref[...] * nl
    xs_ref[N : 2 * N, :] = x1_ref[...] * nl


def _prep_call(degp, x0, x1):
    return pl.pallas_call(
        _prep_body,
        out_shape=jax.ShapeDtypeStruct((2 * N, D), jnp.float32),
    )(degp, x0, x1)


@functools.partial(
    pl.kernel,
    out_type=jax.ShapeDtypeStruct((2 * NP, D), jnp.float32),
    mesh=_MESH,
    scratch_types=[
        pltpu.VMEM_SHARED((NP, D), jnp.float32),
        pltpu.VMEM((2, 16, 128), jnp.int32),
        pltpu.VMEM((2, 16, 128), jnp.int32),
        pltpu.VMEM((128, D), jnp.float32),
        pltpu.VMEM((128, D), jnp.float32),
    ]
    + [pltpu.SemaphoreType.DMA] * 4,
)
def _agg_kernel(xs_hbm, src_hbm, dst_hbm, agg_hbm,
                agg, src_v, dst_v, rows0, rows1, *sems):
    cid = lax.axis_index("c")
    sid = lax.axis_index("s")
    NCH = RC // 16
    SEMS = sems

    def zfill(r, _):
        for k in range(8):
            rows0[r, pl.ds(k * 16, 16)] = jnp.zeros((16,), jnp.float32)
        return _

    lax.fori_loop(0, 128, zfill, None)
    for j in range(NPT // 128):
        pltpu.sync_copy(rows0, agg.at[pl.ds(sid * NPT + j * 128, 128)])
    plsc.subcore_barrier()

    tbase = sid * RC

    def stage(slot, c):
        b = pl.multiple_of(tbase + c * 16, 16)
        pltpu.sync_copy(src_hbm.at[pl.ds(cid * ROWS + b, 16)], src_v.at[slot])
        pltpu.sync_copy(dst_hbm.at[pl.ds(b, 16)], dst_v.at[slot])

    def slot_buf(s):
        return (rows0 if s < 2 else rows1).at[pl.ds((s % 2) * 64, 64)]

    def issue(islot, lr, hf, s):
        pltpu.async_copy(xs_hbm.at[src_v.at[islot, lr, pl.ds(hf * 64, 64)]],
                         slot_buf(s), SEMS[s])

    def wait_g(s):
        pltpu.make_async_copy(xs_hbm.at[pl.ds(0, 64)], slot_buf(s),
                              SEMS[s]).wait()

    def scatter(islot, lr, hf, s):
        pltpu.sync_copy(slot_buf(s),
                        agg.at[dst_v.at[islot, lr, pl.ds(hf * 64, 64)]],
                        add=True)

    stage(0, 0)
    issue(0, 0, 0, 0)
    issue(0, 0, 1, 1)
    issue(0, 1, 0, 2)

    for c in range(NCH):
        p, q = c % 2, (c + 1) % 2
        if c < NCH - 1:
            stage(q, c + 1)

        def block(i, _, p=p):
            for k in range(4):
                lr_t = 2 * i + (k + 3) // 2
                issue(p, lr_t, (k + 3) % 2, (k + 3) % 4)
                wait_g(k)
                scatter(p, 2 * i + k // 2, k % 2, k)
            return _

        lax.fori_loop(0, 7, block, None)
        issue(p, 15, 1, 3)
        wait_g(0)
        scatter(p, 14, 0, 0)
        if c < NCH - 1:
            issue(q, 0, 0, 0)
        wait_g(1)
        scatter(p, 14, 1, 1)
        if c < NCH - 1:
            issue(q, 0, 1, 1)
        wait_g(2)
        scatter(p, 15, 0, 2)
        if c < NCH - 1:
            issue(q, 1, 0, 2)
        wait_g(3)
        scatter(p, 15, 1, 3)
    plsc.subcore_barrier()
    obase = pl.multiple_of(cid * NP + sid * NPT, NPT)
    pltpu.sync_copy(agg.at[pl.ds(sid * NPT, NPT)],
                    agg_hbm.at[pl.ds(obase, NPT)])


def _final_body(degp_ref, agg_ref, wl_ref, bl_ref, wc_ref, bc_ref, out_ref):
    deg = degp_ref[0:N, :] + degp_ref[NP : NP + N, :]
    nl = lax.rsqrt(jnp.maximum(deg, 1.0))
    a0 = agg_ref[0:N, :] * nl
    a1 = agg_ref[NP : NP + N, :] * nl
    r0 = jnp.dot(a0, wl_ref[...], preferred_element_type=jnp.float32)
    r0 = jnp.maximum(r0 + bl_ref[...][None, :], 0.0)
    r1 = jnp.dot(a1, wc_ref[...], preferred_element_type=jnp.float32)
    r1 = jnp.maximum(r1 + bc_ref[...][None, :], 0.0)
    out_ref[...] = r0 + r1


def _final_call(degp, agg, W_l, b_l, W_c, b_c):
    return pl.pallas_call(
        _final_body,
        out_shape=jax.ShapeDtypeStruct((N, D), jnp.float32),
    )(degp, agg, W_l, b_l, W_c, b_c)


def kernel(t, x, edge_index, W_l, b_l, W_c, b_c):
    del t
    x0 = x[:, :, 0]
    x1 = x[:, :, 1]
    ei = edge_index.astype(jnp.int32)
    pad = EP - E
    ar = jnp.arange(pad, dtype=jnp.int32)
    srcp = jnp.concatenate([ei[0], ar % N]).reshape(ROWS, 128)
    dstp = jnp.concatenate([ei[1], N + (ar % (NP - N))]).reshape(ROWS, 128)
    src_all = jnp.concatenate([srcp, srcp + N])

    degp = _deg_kernel(dstp)
    xs = _prep_call(degp, x0, x1)
    agg = _agg_kernel(xs, src_all, dstp)
    return _final_call(degp, agg, W_l, b_l, W_c, b_c)
